# baseline (speedup 1.0000x reference)
import functools

import jax
import jax.numpy as jnp
import numpy as np
from jax import lax
from jax.experimental import pallas as pl
from jax.experimental.pallas import tpu as pltpu
from jax.experimental.pallas import tpu_sc as plsc


_BLK = 1000


def _mm(a, b):
    return jnp.dot(a, b, preferred_element_type=jnp.float32)


def _emit_a(hv, wrel_ref, wroot_ref, bc_ref, hall_ref, root_ref):
    for i in range(wrel_ref.shape[0]):
        hall_ref[i] = _mm(hv, wrel_ref[i]).astype(jnp.bfloat16)
    root_ref[...] = _mm(hv, wroot_ref[...]) + bc_ref[...]


def _unperm_matrix(h):
    p = np.zeros((h, h), np.float32)
    for g in range(h // 32):
        for k in range(16):
            p[32 * g + k, 32 * g + 2 * k] = 1.0
            p[32 * g + 16 + k, 32 * g + 2 * k + 1] = 1.0
    return jnp.asarray(p)


def _pa_body(x_ref, win_ref, bin_ref, wrel_ref, wroot_ref, bc_ref,
             h_ref, hall_ref, root_ref):
    h0 = _mm(x_ref[...], win_ref[...]) + bin_ref[...]
    h_ref[...] = h0
    _emit_a(h0, wrel_ref, wroot_ref, bc_ref, hall_ref, root_ref)


def _tc_pa(x, win, bin_, wrel, wroot, bc):
    n, d = x.shape
    h = win.shape[1]
    r = wrel.shape[0]
    return pl.pallas_call(
        _pa_body,
        grid=(n // _BLK,),
        in_specs=[
            pl.BlockSpec((_BLK, d), lambda i: (i, 0)),
            pl.BlockSpec((d, h), lambda i: (0, 0)),
            pl.BlockSpec((1, h), lambda i: (0, 0)),
            pl.BlockSpec((r, h, h), lambda i: (0, 0, 0)),
            pl.BlockSpec((h, h), lambda i: (0, 0)),
            pl.BlockSpec((1, h), lambda i: (0, 0)),
        ],
        out_specs=[
            pl.BlockSpec((_BLK, h), lambda i: (i, 0)),
            pl.BlockSpec((r, _BLK, h), lambda i: (0, i, 0)),
            pl.BlockSpec((_BLK, h), lambda i: (i, 0)),
        ],
        out_shape=[
            jax.ShapeDtypeStruct((n, h), jnp.float32),
            jax.ShapeDtypeStruct((r, n, h), jnp.bfloat16),
            jax.ShapeDtypeStruct((n, h), jnp.float32),
        ],
    )(x, win, bin_.reshape(1, h), wrel, wroot, bc.reshape(1, h))


def _new_h(agg_ref, root_ref, hprev_ref, g_ref, b_ref, p_ref):
    s = _mm(agg_ref[0] + agg_ref[1], p_ref[...]) + root_ref[...]
    mu = jnp.mean(s, axis=-1, keepdims=True)
    var = jnp.mean((s - mu) ** 2, axis=-1, keepdims=True)
    y = (s - mu) / jnp.sqrt(var + 1e-5) * g_ref[...] + b_ref[...]
    return jnp.maximum(y, 0.0) + hprev_ref[...]


def _ac_body(agg_ref, root_ref, hprev_ref, g_ref, b_ref, p_ref,
             wrel_ref, wroot_ref, bc_ref, h_ref, hall_ref, rootout_ref):
    hnew = _new_h(agg_ref, root_ref, hprev_ref, g_ref, b_ref, p_ref)
    h_ref[...] = hnew
    _emit_a(hnew, wrel_ref, wroot_ref, bc_ref, hall_ref, rootout_ref)


def _tc_ac(agg2, root, hx, g, b, pmat, wrel, wroot, bc):
    n, h = hx.shape
    r = wrel.shape[0]
    return pl.pallas_call(
        _ac_body,
        grid=(n // _BLK,),
        in_specs=[
            pl.BlockSpec((2, _BLK, h), lambda i: (0, i, 0)),
            pl.BlockSpec((_BLK, h), lambda i: (i, 0)),
            pl.BlockSpec((_BLK, h), lambda i: (i, 0)),
            pl.BlockSpec((1, h), lambda i: (0, 0)),
            pl.BlockSpec((1, h), lambda i: (0, 0)),
            pl.BlockSpec((h, h), lambda i: (0, 0)),
            pl.BlockSpec((r, h, h), lambda i: (0, 0, 0)),
            pl.BlockSpec((h, h), lambda i: (0, 0)),
            pl.BlockSpec((1, h), lambda i: (0, 0)),
        ],
        out_specs=[
            pl.BlockSpec((_BLK, h), lambda i: (i, 0)),
            pl.BlockSpec((r, _BLK, h), lambda i: (0, i, 0)),
            pl.BlockSpec((_BLK, h), lambda i: (i, 0)),
        ],
        out_shape=[
            jax.ShapeDtypeStruct((n, h), jnp.float32),
            jax.ShapeDtypeStruct((r, n, h), jnp.bfloat16),
            jax.ShapeDtypeStruct((n, h), jnp.float32),
        ],
    )(agg2, root, hx, g.reshape(1, h), b.reshape(1, h), pmat,
      wrel, wroot, bc.reshape(1, h))


def _cpool_body(agg_ref, root_ref, hprev_ref, g_ref, b_ref, p_ref,
                batch_ref, o_ref):
    hnew = _new_h(agg_ref, root_ref, hprev_ref, g_ref, b_ref, p_ref)
    ng = o_ref.shape[0]
    blk = hnew.shape[0]

    @pl.when(pl.program_id(0) == 0)
    def _():
        o_ref[...] = jnp.zeros_like(o_ref)

    bvec = batch_ref[0]
    onehot = (
        bvec == lax.broadcasted_iota(jnp.int32, (ng, blk), 0)
    ).astype(jnp.float32)
    o_ref[...] += lax.dot_general(
        onehot, hnew, (((1,), (0,)), ((), ())),
        preferred_element_type=jnp.float32,
    )


def _tc_cpool(agg2, root, hx, g, b, pmat, batch3d, ng):
    n, h = hx.shape
    return pl.pallas_call(
        _cpool_body,
        grid=(n // _BLK,),
        in_specs=[
            pl.BlockSpec((2, _BLK, h), lambda i: (0, i, 0)),
            pl.BlockSpec((_BLK, h), lambda i: (i, 0)),
            pl.BlockSpec((_BLK, h), lambda i: (i, 0)),
            pl.BlockSpec((1, h), lambda i: (0, 0)),
            pl.BlockSpec((1, h), lambda i: (0, 0)),
            pl.BlockSpec((h, h), lambda i: (0, 0)),
            pl.BlockSpec((1, 1, _BLK), lambda i: (i, 0, 0)),
        ],
        out_specs=pl.BlockSpec((ng, h), lambda i: (0, 0)),
        out_shape=jax.ShapeDtypeStruct((ng, h), jnp.float32),
    )(agg2, root, hx, g.reshape(1, h), b.reshape(1, h), pmat, batch3d)


def _inv_body(cnt_ref, o_ref):
    c = cnt_ref[0] + cnt_ref[1]
    o_ref[...] = jnp.where(c > 0, 1.0 / jnp.maximum(c, 1.0), 0.0)


def _tc_inv(cnt2_3d):
    _, rows, cols = cnt2_3d.shape
    return pl.pallas_call(
        _inv_body,
        grid=(1,),
        in_specs=[pl.BlockSpec((2, rows, cols), lambda i: (0, 0, 0))],
        out_specs=pl.BlockSpec((rows, cols), lambda i: (0, 0)),
        out_shape=jax.ShapeDtypeStruct((rows, cols), jnp.float32),
    )(cnt2_3d)


_LANES = 16


def _zero_fill(ref, nelem):
    z = jnp.zeros((_LANES,), jnp.float32)

    def body(i, _):
        ref[pl.ds(i * _LANES, _LANES)] = z
        return 0

    lax.fori_loop(0, nelem // _LANES, body, 0, unroll=4)


def _sc_count(key16, nk, nc, ns):
    e = key16.shape[0]
    nw = nc * ns
    per_w = e // nw
    ck = 2000
    n_chunks = per_w // ck
    per_tile = nk // ns
    mesh = plsc.VectorSubcoreMesh(core_axis_name="c", subcore_axis_name="s")

    @functools.partial(
        pl.kernel,
        out_type=jax.ShapeDtypeStruct((2 * nk,), jnp.float32),
        mesh=mesh,
        scratch_types=[
            pltpu.VMEM((ck,), jnp.int32),
            pltpu.VMEM((ck,), jnp.float32),
            pltpu.VMEM((per_tile,), jnp.float32),
            pltpu.VMEM_SHARED((nk,), jnp.float32),
            pltpu.SemaphoreType.DMA,
        ],
    )
    def k(key_hbm, out_hbm, key_v, ones_v, zb, cnt_sp, sem):
        cid = lax.axis_index("c")
        sid = lax.axis_index("s")
        wid = sid * nc + cid

        o = jnp.ones((_LANES,), jnp.float32)

        def fill_ones(i, _):
            ones_v[pl.ds(i * _LANES, _LANES)] = o
            return 0

        lax.fori_loop(0, ck // _LANES, fill_ones, 0, unroll=4)

        _zero_fill(zb, per_tile)
        pltpu.sync_copy(zb, cnt_sp.at[pl.ds(sid * per_tile, per_tile)])
        plsc.subcore_barrier()

        def chunk(i, _):
            base = wid * per_w + i * ck
            pltpu.sync_copy(key_hbm.at[pl.ds(base, ck)], key_v)
            pltpu.sync_copy(ones_v, cnt_sp.at[key_v], add=True)
            return 0

        lax.fori_loop(0, n_chunks, chunk, 0)
        plsc.subcore_barrier()

        pltpu.sync_copy(cnt_sp.at[pl.ds(sid * per_tile, per_tile)], zb)
        pltpu.sync_copy(zb, out_hbm.at[pl.ds(cid * nk + sid * per_tile, per_tile)])

    return k(key16).reshape(2, nk)


def _sc_norm_gather(invflat, key16, nc, ns):
    e = key16.shape[0]
    nw = nc * ns
    per_w = e // nw
    ck = 2000
    n_chunks = per_w // ck
    mesh = plsc.VectorSubcoreMesh(core_axis_name="c", subcore_axis_name="s")

    @functools.partial(
        pl.kernel,
        out_type=jax.ShapeDtypeStruct((e,), jnp.float32),
        mesh=mesh,
        scratch_types=[
            pltpu.VMEM((ck,), jnp.int32),
            pltpu.VMEM((ck,), jnp.float32),
            pltpu.SemaphoreType.DMA,
        ],
    )
    def k(inv_hbm, key_hbm, out_hbm, key_v, nv, sem):
        cid = lax.axis_index("c")
        sid = lax.axis_index("s")
        wid = sid * nc + cid

        def chunk(i, _):
            base = wid * per_w + i * ck
            pltpu.sync_copy(key_hbm.at[pl.ds(base, ck)], key_v)
            pltpu.async_copy(inv_hbm.at[key_v], nv, sem).wait()
            pltpu.sync_copy(nv, out_hbm.at[pl.ds(base, ck)])
            return 0

        lax.fori_loop(0, n_chunks, chunk, 0)

    return k(invflat, key16)


def _sc_layer_agg(hall_flat, idx, dst, norm, n, h, nc, ns):
    e = idx.shape[0]
    nw = nc * ns
    per_w = e // nw
    ek = 80
    n_chunks = per_w // ek
    n_pairs = (n_chunks - 1) // 2
    zrows = ek
    n_blocks = n // zrows
    mesh = plsc.VectorSubcoreMesh(core_axis_name="c", subcore_axis_name="s")

    idx2 = idx.reshape(nw, per_w)
    dst2 = dst.reshape(nw, per_w)
    norm2 = norm.reshape(nw, per_w)

    @functools.partial(
        pl.kernel,
        out_type=jax.ShapeDtypeStruct((2, n, h), jnp.float32),
        mesh=mesh,
        compiler_params=pltpu.CompilerParams(
            needs_layout_passes=False, use_tc_tiling_on_sc=False
        ),
        scratch_types=[
            pltpu.VMEM((per_w,), jnp.int32),
            pltpu.VMEM((per_w,), jnp.float32),
            pltpu.VMEM((2, ek, h // 2), jnp.int32),
            pltpu.VMEM((2, ek, h), jnp.float32),
            pltpu.VMEM((ek,), jnp.int32),
            pltpu.VMEM((ek,), jnp.int32),
            pltpu.VMEM_SHARED((n, h), jnp.float32),
            pltpu.SemaphoreType.DMA,
            pltpu.SemaphoreType.DMA,
            pltpu.SemaphoreType.DMA,
            pltpu.SemaphoreType.DMA,
            pltpu.SemaphoreType.DMA,
            pltpu.SemaphoreType.DMA,
        ],
    )
    def k(hall_hbm, idx_hbm, dst_hbm, norm_hbm, out_hbm,
          idx_v, norm_v, rows_i, rows_f, dstc0, dstc1, agg_sp,
          gsem0, gsem1, ssem0, ssem1, dsem0, dsem1):
        cid = lax.axis_index("c")
        sid = lax.axis_index("s")
        wid = sid * nc + cid
        gsems = (gsem0, gsem1)
        ssems = (ssem0, ssem1)
        dsems = (dsem0, dsem1)
        dstcs = (dstc0, dstc1)

        pltpu.async_copy(idx_hbm.at[wid], idx_v, gsem0)
        pltpu.async_copy(norm_hbm.at[wid], norm_v, gsem0)

        z = jnp.zeros((_LANES,), jnp.float32)

        def zfill(i, _):
            r = i // (h // _LANES)
            c = i % (h // _LANES)
            rows_f[0, r, pl.ds(c * _LANES, _LANES)] = z
            return 0

        lax.fori_loop(0, zrows * (h // _LANES), zfill, 0, unroll=4)
        zsrc = rows_f.at[0]
        for j in range((n_blocks + ns - 1) // ns):
            blk = j * ns + sid

            @pl.when(blk < n_blocks)
            def _():
                pltpu.sync_copy(zsrc, agg_sp.at[pl.ds(blk * zrows, zrows)])
        pltpu.make_async_copy(idx_hbm.at[wid], idx_v, gsem0).wait()
        pltpu.make_async_copy(norm_hbm.at[wid], norm_v, gsem0).wait()
        plsc.subcore_barrier()

        def g_start(c, b):
            pltpu.async_copy(
                hall_hbm.at[idx_v.at[pl.ds(c * ek, ek)]], rows_i.at[b], gsems[b]
            )

        def g_wait(b):
            pltpu.make_async_copy(
                hall_hbm.at[idx_v.at[pl.ds(0, ek)]], rows_i.at[b], gsems[b]
            ).wait()

        def d_start(c, b):
            pltpu.async_copy(
                dst_hbm.at[pl.ds(wid * per_w + c * ek, ek)], dstcs[b], dsems[b]
            )

        def d_wait(b):
            pltpu.make_async_copy(
                dst_hbm.at[pl.ds(0, ek)], dstcs[b], dsems[b]
            ).wait()

        himask = jnp.full((_LANES,), -65536, jnp.int32)
        shl16 = jnp.full((_LANES,), 16, jnp.int32)

        def scale(c, b):
            def grp(gi, _):
                nv16 = norm_v[pl.ds(c * ek + gi * _LANES, _LANES)]
                for l in range(_LANES):
                    ei = gi * _LANES + l
                    nb = jnp.broadcast_to(nv16[l], (_LANES,))
                    for g4 in range(h // 32):
                        w = rows_i[b, ei, pl.ds(_LANES * g4, _LANES)]
                        lo = plsc.bitcast(lax.shift_left(w, shl16), jnp.float32)
                        hi = plsc.bitcast(w & himask, jnp.float32)
                        rows_f[b, ei, pl.ds(32 * g4, _LANES)] = lo * nb
                        rows_f[b, ei, pl.ds(32 * g4 + _LANES, _LANES)] = hi * nb
                return 0

            lax.fori_loop(0, ek // _LANES, grp, 0)

        def s_start(c, b):
            d_wait(b)
            pltpu.async_copy(rows_f.at[b], agg_sp.at[dstcs[b]], ssems[b], add=True)

        def s_wait(b):
            pltpu.make_async_copy(
                rows_f.at[b], agg_sp.at[dstcs[b]], ssems[b]
            ).wait()

        d_start(0, 0)
        g_start(0, 0)

        def pair(j, _):
            c0 = 2 * j

            @pl.when(j > 0)
            def _():
                s_wait(1)

            g_start(c0 + 1, 1)
            d_start(c0 + 1, 1)
            g_wait(0)
            scale(c0, 0)
            s_start(c0, 0)
            s_wait(0)
            g_start(c0 + 2, 0)
            d_start(c0 + 2, 0)
            g_wait(1)
            scale(c0 + 1, 1)
            s_start(c0 + 1, 1)
            return 0

        lax.fori_loop(0, n_pairs, pair, 0)
        s_wait(1)
        g_wait(0)
        scale(n_chunks - 1, 0)
        s_start(n_chunks - 1, 0)
        s_wait(0)
        plsc.subcore_barrier()

        njj = (n_blocks + ns - 1) // ns
        for j in range(njj):
            blk = j * ns + sid
            b = j % 2

            @pl.when(blk < n_blocks)
            def _():
                if j >= 2:
                    pltpu.make_async_copy(
                        rows_f.at[b],
                        out_hbm.at[cid, pl.ds((j - 2) * ns * zrows, zrows)],
                        gsems[b],
                    ).wait()
                r0 = blk * zrows
                pltpu.sync_copy(agg_sp.at[pl.ds(r0, zrows)], rows_f.at[b])
                pltpu.async_copy(
                    rows_f.at[b], out_hbm.at[cid, pl.ds(r0, zrows)], gsems[b]
                )
        for j in range(max(njj - 2, 0), njj):
            blk = j * ns + sid
            b = j % 2

            @pl.when(blk < n_blocks)
            def _():
                pltpu.make_async_copy(
                    rows_f.at[b],
                    out_hbm.at[cid, pl.ds(blk * zrows, zrows)],
                    gsems[b],
                ).wait()

    return k(hall_flat, idx2, dst, norm2)


def kernel(x, edge_index, edge_type, batch, W_in, b_in, W_rel, W_root,
           b_conv, ln_g, ln_b):
    n, d = x.shape
    hdim = W_in.shape[1]
    nlayers, r = W_rel.shape[0], W_rel.shape[1]
    g = 16
    info = plsc.get_sparse_core_info()
    nc, ns = info.num_cores, info.num_subcores

    src = edge_index[0]
    dst = edge_index[1]
    idx = edge_type * n + src
    key16 = dst * 16 + edge_type

    nk = n * 16
    cnt2 = _sc_count(key16, nk, nc, ns)
    inv = _tc_inv(cnt2.reshape(2, nk // 128, 128))
    norm = _sc_norm_gather(inv.reshape(nk), key16, nc, ns)

    batch3d = batch.reshape(n // _BLK, 1, _BLK)
    pmat = _unperm_matrix(hdim)
    h, h_all, root = _tc_pa(x, W_in, b_in, W_rel[0], W_root[0], b_conv[0])
    for l in range(nlayers):
        hall_i = lax.bitcast_convert_type(
            h_all.reshape(r * n, hdim // 2, 2), jnp.int32
        )
        agg2 = _sc_layer_agg(hall_i, idx, dst, norm, n, hdim, nc, ns)
        if l + 1 < nlayers:
            h, h_all, root = _tc_ac(
                agg2, root, h, ln_g[l], ln_b[l], pmat,
                W_rel[l + 1], W_root[l + 1], b_conv[l + 1],
            )
    return _tc_cpool(
        agg2, root, h, ln_g[nlayers - 1], ln_b[nlayers - 1], pmat, batch3d, g
    )


# speedup vs baseline: 4.2140x; 4.2140x over previous
import functools

import jax
import jax.numpy as jnp
import numpy as np
from jax import lax
from jax.experimental import pallas as pl
from jax.experimental.pallas import tpu as pltpu
from jax.experimental.pallas import tpu_sc as plsc


_BLK = 1000


def _mm(a, b):
    return jnp.dot(a, b, preferred_element_type=jnp.float32)


def _emit_a(hv, wrel_ref, wroot_ref, bc_ref, hall_ref, root_ref):
    for i in range(wrel_ref.shape[0]):
        hall_ref[i] = _mm(hv, wrel_ref[i])
    root_ref[...] = _mm(hv, wroot_ref[...]) + bc_ref[...]


def _pa_body(x_ref, win_ref, bin_ref, wrel_ref, wroot_ref, bc_ref,
             h_ref, hall_ref, root_ref):
    h0 = _mm(x_ref[...], win_ref[...]) + bin_ref[...]
    h_ref[...] = h0
    _emit_a(h0, wrel_ref, wroot_ref, bc_ref, hall_ref, root_ref)


def _tc_pa(x, win, bin_, wrel, wroot, bc):
    n, d = x.shape
    h = win.shape[1]
    r = wrel.shape[0]
    return pl.pallas_call(
        _pa_body,
        grid=(n // _BLK,),
        in_specs=[
            pl.BlockSpec((_BLK, d), lambda i: (i, 0)),
            pl.BlockSpec((d, h), lambda i: (0, 0)),
            pl.BlockSpec((1, h), lambda i: (0, 0)),
            pl.BlockSpec((r, h, h), lambda i: (0, 0, 0)),
            pl.BlockSpec((h, h), lambda i: (0, 0)),
            pl.BlockSpec((1, h), lambda i: (0, 0)),
        ],
        out_specs=[
            pl.BlockSpec((_BLK, h), lambda i: (i, 0)),
            pl.BlockSpec((r, _BLK, h), lambda i: (0, i, 0)),
            pl.BlockSpec((_BLK, h), lambda i: (i, 0)),
        ],
        out_shape=[
            jax.ShapeDtypeStruct((n, h), jnp.float32),
            jax.ShapeDtypeStruct((r, n, h), jnp.float32),
            jax.ShapeDtypeStruct((n, h), jnp.float32),
        ],
    )(x, win, bin_.reshape(1, h), wrel, wroot, bc.reshape(1, h))


def _new_h(agg_ref, root_ref, hprev_ref, g_ref, b_ref):
    s = agg_ref[0] + agg_ref[1] + root_ref[...]
    mu = jnp.mean(s, axis=-1, keepdims=True)
    var = jnp.mean((s - mu) ** 2, axis=-1, keepdims=True)
    y = (s - mu) / jnp.sqrt(var + 1e-5) * g_ref[...] + b_ref[...]
    return jnp.maximum(y, 0.0) + hprev_ref[...]


def _ac_body(agg_ref, root_ref, hprev_ref, g_ref, b_ref,
             wrel_ref, wroot_ref, bc_ref, h_ref, hall_ref, rootout_ref):
    hnew = _new_h(agg_ref, root_ref, hprev_ref, g_ref, b_ref)
    h_ref[...] = hnew
    _emit_a(hnew, wrel_ref, wroot_ref, bc_ref, hall_ref, rootout_ref)


def _tc_ac(agg2, root, hx, g, b, wrel, wroot, bc):
    n, h = hx.shape
    r = wrel.shape[0]
    return pl.pallas_call(
        _ac_body,
        grid=(n // _BLK,),
        in_specs=[
            pl.BlockSpec((2, _BLK, h), lambda i: (0, i, 0)),
            pl.BlockSpec((_BLK, h), lambda i: (i, 0)),
            pl.BlockSpec((_BLK, h), lambda i: (i, 0)),
            pl.BlockSpec((1, h), lambda i: (0, 0)),
            pl.BlockSpec((1, h), lambda i: (0, 0)),
            pl.BlockSpec((r, h, h), lambda i: (0, 0, 0)),
            pl.BlockSpec((h, h), lambda i: (0, 0)),
            pl.BlockSpec((1, h), lambda i: (0, 0)),
        ],
        out_specs=[
            pl.BlockSpec((_BLK, h), lambda i: (i, 0)),
            pl.BlockSpec((r, _BLK, h), lambda i: (0, i, 0)),
            pl.BlockSpec((_BLK, h), lambda i: (i, 0)),
        ],
        out_shape=[
            jax.ShapeDtypeStruct((n, h), jnp.float32),
            jax.ShapeDtypeStruct((r, n, h), jnp.float32),
            jax.ShapeDtypeStruct((n, h), jnp.float32),
        ],
    )(agg2, root, hx, g.reshape(1, h), b.reshape(1, h),
      wrel, wroot, bc.reshape(1, h))


def _cpool_body(agg_ref, root_ref, hprev_ref, g_ref, b_ref,
                batch_ref, o_ref):
    hnew = _new_h(agg_ref, root_ref, hprev_ref, g_ref, b_ref)
    ng = o_ref.shape[0]
    blk = hnew.shape[0]

    @pl.when(pl.program_id(0) == 0)
    def _():
        o_ref[...] = jnp.zeros_like(o_ref)

    bvec = batch_ref[0]
    onehot = (
        bvec == lax.broadcasted_iota(jnp.int32, (ng, blk), 0)
    ).astype(jnp.float32)
    o_ref[...] += lax.dot_general(
        onehot, hnew, (((1,), (0,)), ((), ())),
        preferred_element_type=jnp.float32,
    )


def _tc_cpool(agg2, root, hx, g, b, batch3d, ng):
    n, h = hx.shape
    return pl.pallas_call(
        _cpool_body,
        grid=(n // _BLK,),
        in_specs=[
            pl.BlockSpec((2, _BLK, h), lambda i: (0, i, 0)),
            pl.BlockSpec((_BLK, h), lambda i: (i, 0)),
            pl.BlockSpec((_BLK, h), lambda i: (i, 0)),
            pl.BlockSpec((1, h), lambda i: (0, 0)),
            pl.BlockSpec((1, h), lambda i: (0, 0)),
            pl.BlockSpec((1, 1, _BLK), lambda i: (i, 0, 0)),
        ],
        out_specs=pl.BlockSpec((ng, h), lambda i: (0, 0)),
        out_shape=jax.ShapeDtypeStruct((ng, h), jnp.float32),
    )(agg2, root, hx, g.reshape(1, h), b.reshape(1, h), batch3d)


def _inv_body(cnt_ref, o_ref):
    c = cnt_ref[0] + cnt_ref[1]
    o_ref[...] = jnp.where(c > 0, 1.0 / jnp.maximum(c, 1.0), 0.0)


def _tc_inv(cnt2_3d):
    _, rows, cols = cnt2_3d.shape
    return pl.pallas_call(
        _inv_body,
        grid=(1,),
        in_specs=[pl.BlockSpec((2, rows, cols), lambda i: (0, 0, 0))],
        out_specs=pl.BlockSpec((rows, cols), lambda i: (0, 0)),
        out_shape=jax.ShapeDtypeStruct((rows, cols), jnp.float32),
    )(cnt2_3d)


_LANES = 16


def _zero_fill(ref, nelem):
    z = jnp.zeros((_LANES,), jnp.float32)

    def body(i, _):
        ref[pl.ds(i * _LANES, _LANES)] = z
        return 0

    lax.fori_loop(0, nelem // _LANES, body, 0, unroll=4)


def _sc_count(key16, nk, nc, ns):
    e = key16.shape[0]
    nw = nc * ns
    per_w = e // nw
    ck = 2000
    n_chunks = per_w // ck
    per_tile = nk // ns
    mesh = plsc.VectorSubcoreMesh(core_axis_name="c", subcore_axis_name="s")

    @functools.partial(
        pl.kernel,
        out_type=jax.ShapeDtypeStruct((2 * nk,), jnp.float32),
        mesh=mesh,
        scratch_types=[
            pltpu.VMEM((ck,), jnp.int32),
            pltpu.VMEM((ck,), jnp.float32),
            pltpu.VMEM((per_tile,), jnp.float32),
            pltpu.VMEM_SHARED((nk,), jnp.float32),
            pltpu.SemaphoreType.DMA,
        ],
    )
    def k(key_hbm, out_hbm, key_v, ones_v, zb, cnt_sp, sem):
        cid = lax.axis_index("c")
        sid = lax.axis_index("s")
        wid = sid * nc + cid

        o = jnp.ones((_LANES,), jnp.float32)

        def fill_ones(i, _):
            ones_v[pl.ds(i * _LANES, _LANES)] = o
            return 0

        lax.fori_loop(0, ck // _LANES, fill_ones, 0, unroll=4)

        _zero_fill(zb, per_tile)
        pltpu.sync_copy(zb, cnt_sp.at[pl.ds(sid * per_tile, per_tile)])
        plsc.subcore_barrier()

        def chunk(i, _):
            base = wid * per_w + i * ck
            pltpu.sync_copy(key_hbm.at[pl.ds(base, ck)], key_v)
            pltpu.sync_copy(ones_v, cnt_sp.at[key_v], add=True)
            return 0

        lax.fori_loop(0, n_chunks, chunk, 0)
        plsc.subcore_barrier()

        pltpu.sync_copy(cnt_sp.at[pl.ds(sid * per_tile, per_tile)], zb)
        pltpu.sync_copy(zb, out_hbm.at[pl.ds(cid * nk + sid * per_tile, per_tile)])

    return k(key16).reshape(2, nk)


def _sc_norm_gather(invflat, key16, nc, ns):
    e = key16.shape[0]
    nw = nc * ns
    per_w = e // nw
    ck = 2000
    n_chunks = per_w // ck
    mesh = plsc.VectorSubcoreMesh(core_axis_name="c", subcore_axis_name="s")

    @functools.partial(
        pl.kernel,
        out_type=jax.ShapeDtypeStruct((e,), jnp.float32),
        mesh=mesh,
        scratch_types=[
            pltpu.VMEM((ck,), jnp.int32),
            pltpu.VMEM((ck,), jnp.float32),
            pltpu.SemaphoreType.DMA,
        ],
    )
    def k(inv_hbm, key_hbm, out_hbm, key_v, nv, sem):
        cid = lax.axis_index("c")
        sid = lax.axis_index("s")
        wid = sid * nc + cid

        def chunk(i, _):
            base = wid * per_w + i * ck
            pltpu.sync_copy(key_hbm.at[pl.ds(base, ck)], key_v)
            pltpu.async_copy(inv_hbm.at[key_v], nv, sem).wait()
            pltpu.sync_copy(nv, out_hbm.at[pl.ds(base, ck)])
            return 0

        lax.fori_loop(0, n_chunks, chunk, 0)

    return k(invflat, key16)


def _sc_layer_agg(hall_flat, idx, dst, norm, n, h, nc, ns):
    e = idx.shape[0]
    nw = nc * ns
    per_w = e // nw
    ek = 80
    n_chunks = per_w // ek
    n_pairs = (n_chunks - 1) // 2
    zrows = ek
    n_blocks = n // zrows
    mesh = plsc.VectorSubcoreMesh(core_axis_name="c", subcore_axis_name="s")

    idx2 = idx.reshape(nw, per_w)
    dst2 = dst.reshape(nw, per_w)
    norm2 = norm.reshape(nw, per_w)

    @functools.partial(
        pl.kernel,
        out_type=jax.ShapeDtypeStruct((2, n, h), jnp.float32),
        mesh=mesh,
        scratch_types=[
            pltpu.VMEM((per_w,), jnp.int32),
            pltpu.VMEM((per_w,), jnp.int32),
            pltpu.VMEM((per_w,), jnp.float32),
            pltpu.VMEM((2, ek, h), jnp.float32),
            pltpu.VMEM((ek,), jnp.int32),
            pltpu.VMEM((ek,), jnp.int32),
            pltpu.VMEM_SHARED((n, h), jnp.float32),
            pltpu.SemaphoreType.DMA,
            pltpu.SemaphoreType.DMA,
            pltpu.SemaphoreType.DMA,
            pltpu.SemaphoreType.DMA,
        ],
    )
    def k(hall_hbm, idx_hbm, dst_hbm, norm_hbm, out_hbm,
          idx_v, dst_v, norm_v, rows_f, dstc0, dstc1, agg_sp,
          gsem0, gsem1, ssem0, ssem1):
        cid = lax.axis_index("c")
        sid = lax.axis_index("s")
        wid = sid * nc + cid
        gsems = (gsem0, gsem1)
        ssems = (ssem0, ssem1)
        dstcs = (dstc0, dstc1)

        pltpu.async_copy(idx_hbm.at[wid], idx_v, gsem0)
        pltpu.async_copy(dst_hbm.at[wid], dst_v, gsem0)
        pltpu.async_copy(norm_hbm.at[wid], norm_v, gsem0)

        z = jnp.zeros((_LANES,), jnp.float32)

        def zfill(i, _):
            r = i // (h // _LANES)
            c = i % (h // _LANES)
            rows_f[0, r, pl.ds(c * _LANES, _LANES)] = z
            return 0

        lax.fori_loop(0, zrows * (h // _LANES), zfill, 0, unroll=4)
        zsrc = rows_f.at[0]
        for j in range((n_blocks + ns - 1) // ns):
            blk = j * ns + sid

            @pl.when(blk < n_blocks)
            def _():
                pltpu.sync_copy(zsrc, agg_sp.at[pl.ds(blk * zrows, zrows)])
        pltpu.make_async_copy(idx_hbm.at[wid], idx_v, gsem0).wait()
        pltpu.make_async_copy(dst_hbm.at[wid], dst_v, gsem0).wait()
        pltpu.make_async_copy(norm_hbm.at[wid], norm_v, gsem0).wait()
        plsc.subcore_barrier()

        def g_start(c, b):
            pltpu.async_copy(
                hall_hbm.at[idx_v.at[pl.ds(c * ek, ek)]], rows_f.at[b], gsems[b]
            )

        def g_wait(b):
            pltpu.make_async_copy(
                hall_hbm.at[idx_v.at[pl.ds(0, ek)]], rows_f.at[b], gsems[b]
            ).wait()

        def scale(c, b):
            def grp(gi, _):
                nv16 = norm_v[pl.ds(c * ek + gi * _LANES, _LANES)]
                for l in range(_LANES):
                    ei = gi * _LANES + l
                    nb = jnp.broadcast_to(nv16[l], (_LANES,))
                    for cc in range(h // _LANES):
                        sl = pl.ds(cc * _LANES, _LANES)
                        rows_f[b, ei, sl] = rows_f[b, ei, sl] * nb
                return 0

            lax.fori_loop(0, ek // _LANES, grp, 0)

        def s_start(c, b):
            dstc = dstcs[b]
            for g in range(ek // _LANES):
                sl = pl.ds(g * _LANES, _LANES)
                dstc[sl] = dst_v[pl.ds(c * ek + g * _LANES, _LANES)]
            pltpu.async_copy(rows_f.at[b], agg_sp.at[dstc], ssems[b], add=True)

        def s_wait(b):
            pltpu.make_async_copy(
                rows_f.at[b], agg_sp.at[dstcs[b]], ssems[b]
            ).wait()

        g_start(0, 0)

        def pair(j, _):
            c0 = 2 * j
            g_wait(0)

            @pl.when(j > 0)
            def _():
                s_wait(1)

            g_start(c0 + 1, 1)
            scale(c0, 0)
            s_start(c0, 0)
            g_wait(1)
            s_wait(0)
            g_start(c0 + 2, 0)
            scale(c0 + 1, 1)
            s_start(c0 + 1, 1)
            return 0

        lax.fori_loop(0, n_pairs, pair, 0)
        g_wait(0)
        s_wait(1)
        scale(n_chunks - 1, 0)
        s_start(n_chunks - 1, 0)
        s_wait(0)
        plsc.subcore_barrier()

        njj = (n_blocks + ns - 1) // ns
        for j in range(njj):
            blk = j * ns + sid
            b = j % 2

            @pl.when(blk < n_blocks)
            def _():
                if j >= 2:
                    pltpu.make_async_copy(
                        rows_f.at[b],
                        out_hbm.at[cid, pl.ds((j - 2) * ns * zrows, zrows)],
                        gsems[b],
                    ).wait()
                r0 = blk * zrows
                pltpu.sync_copy(agg_sp.at[pl.ds(r0, zrows)], rows_f.at[b])
                pltpu.async_copy(
                    rows_f.at[b], out_hbm.at[cid, pl.ds(r0, zrows)], gsems[b]
                )
        for j in range(max(njj - 2, 0), njj):
            blk = j * ns + sid
            b = j % 2

            @pl.when(blk < n_blocks)
            def _():
                pltpu.make_async_copy(
                    rows_f.at[b],
                    out_hbm.at[cid, pl.ds(blk * zrows, zrows)],
                    gsems[b],
                ).wait()

    return k(hall_flat, idx2, dst2, norm2)


def kernel(x, edge_index, edge_type, batch, W_in, b_in, W_rel, W_root,
           b_conv, ln_g, ln_b):
    n, d = x.shape
    hdim = W_in.shape[1]
    nlayers, r = W_rel.shape[0], W_rel.shape[1]
    g = 16
    info = plsc.get_sparse_core_info()
    nc, ns = info.num_cores, info.num_subcores

    src = edge_index[0]
    dst = edge_index[1]
    idx = edge_type * n + src
    key16 = dst * 16 + edge_type

    nk = n * 16
    cnt2 = _sc_count(key16, nk, nc, ns)
    inv = _tc_inv(cnt2.reshape(2, nk // 128, 128))
    norm = _sc_norm_gather(inv.reshape(nk), key16, nc, ns)

    batch3d = batch.reshape(n // _BLK, 1, _BLK)
    h, h_all, root = _tc_pa(x, W_in, b_in, W_rel[0], W_root[0], b_conv[0])
    for l in range(nlayers):
        agg2 = _sc_layer_agg(
            h_all.reshape(r * n, hdim), idx, dst, norm, n, hdim, nc, ns
        )
        if l + 1 < nlayers:
            h, h_all, root = _tc_ac(
                agg2, root, h, ln_g[l], ln_b[l],
                W_rel[l + 1], W_root[l + 1], b_conv[l + 1],
            )
    return _tc_cpool(
        agg2, root, h, ln_g[nlayers - 1], ln_b[nlayers - 1], batch3d, g
    )
